# initial kernel scaffold (unmeasured)
import jax
import jax.numpy as jnp
from jax import lax
from jax.experimental import pallas as pl
from jax.experimental.pallas import tpu as pltpu

N_DEV = 8


def kernel(x, W):
    t, d = x.shape
    _, v_loc = W.shape
    v_tot = N_DEV * v_loc

    def body(x_ref, w_ref, out_ref, comm_ref, send_sems, recv_sems):
        my = lax.axis_index("i")
        left = lax.rem(my + (N_DEV - 1), N_DEV)
        right = lax.rem(my + 1, N_DEV)

        barrier_sem = pltpu.get_barrier_semaphore()
        for nbr in (left, right):
            pl.semaphore_signal(
                barrier_sem, inc=1,
                device_id=(nbr,), device_id_type=pl.DeviceIdType.MESH,
            )
        pl.semaphore_wait(barrier_sem, 2)

        xb = x_ref[...].astype(jnp.bfloat16)
        wb = w_ref[...].astype(jnp.bfloat16)
        logits = jnp.dot(xb, wb, preferred_element_type=jnp.float32)
        comm_ref[0] = logits.astype(jnp.bfloat16)

        for h in range(N_DEV - 1):
            rdma = pltpu.make_async_remote_copy(
                src_ref=comm_ref.at[h],
                dst_ref=comm_ref.at[h + 1],
                send_sem=send_sems.at[h],
                recv_sem=recv_sems.at[h + 1],
                device_id=(right,),
                device_id_type=pl.DeviceIdType.MESH,
            )
            rdma.start()
            rdma.wait()

        m = jnp.full((t, 1), -jnp.inf, dtype=jnp.float32)
        for k in range(N_DEV):
            lk = comm_ref[k].astype(jnp.float32)
            m = jnp.maximum(m, jnp.max(lk, axis=1, keepdims=True))
        s = jnp.zeros((t, 1), dtype=jnp.float32)
        for k in range(N_DEV):
            lk = comm_ref[k].astype(jnp.float32)
            s = s + jnp.sum(jnp.exp(lk - m), axis=1, keepdims=True)
        inv = 1.0 / s
        for k in range(N_DEV):
            origin = lax.rem(my - k + N_DEV, N_DEV)
            lk = comm_ref[k].astype(jnp.float32)
            out_ref[:, pl.ds(origin * v_loc, v_loc)] = jnp.exp(lk - m) * inv

    return pl.pallas_call(
        body,
        out_shape=jax.ShapeDtypeStruct((t, v_tot), jnp.float32),
        in_specs=[
            pl.BlockSpec(memory_space=pltpu.VMEM),
            pl.BlockSpec(memory_space=pltpu.VMEM),
        ],
        out_specs=pl.BlockSpec(memory_space=pltpu.VMEM),
        scratch_shapes=[
            pltpu.VMEM((N_DEV, t, v_loc), jnp.bfloat16),
            pltpu.SemaphoreType.DMA((N_DEV,)),
            pltpu.SemaphoreType.DMA((N_DEV,)),
        ],
        compiler_params=pltpu.CompilerParams(collective_id=0),
    )(x, W)


# baseline (device time: 229716 ns/iter reference)
import jax
import jax.numpy as jnp
from jax import lax
from jax.experimental import pallas as pl
from jax.experimental.pallas import tpu as pltpu

N_DEV = 8


def kernel(x, W):
    t, d = x.shape
    _, v_loc = W.shape
    v_tot = N_DEV * v_loc

    def body(x_ref, w_ref, out_ref, comm_ref, stage_ref,
             send_sems, recv_sems, out_sems):
        my = lax.axis_index("i")
        left = lax.rem(my + (N_DEV - 1), N_DEV)
        right = lax.rem(my + 1, N_DEV)

        barrier_sem = pltpu.get_barrier_semaphore()
        for nbr in (left, right):
            pl.semaphore_signal(
                barrier_sem, inc=1,
                device_id=(nbr,), device_id_type=pl.DeviceIdType.MESH,
            )
        pl.semaphore_wait(barrier_sem, 2)

        xb = x_ref[...].astype(jnp.bfloat16)
        wb = w_ref[...].astype(jnp.bfloat16)
        logits = jnp.dot(xb, wb, preferred_element_type=jnp.float32)
        comm_ref[0] = logits.astype(jnp.bfloat16)

        for h in range(N_DEV - 1):
            rdma = pltpu.make_async_remote_copy(
                src_ref=comm_ref.at[h],
                dst_ref=comm_ref.at[h + 1],
                send_sem=send_sems.at[h],
                recv_sem=recv_sems.at[h + 1],
                device_id=(right,),
                device_id_type=pl.DeviceIdType.MESH,
            )
            rdma.start()
            rdma.wait()

        def max_body(k, mb):
            return jnp.maximum(mb, comm_ref[k])

        mb = lax.fori_loop(1, N_DEV, max_body, comm_ref[0])
        m = jnp.max(mb.astype(jnp.float32), axis=1, keepdims=True)

        def exp_body(k, s):
            ek = jnp.exp(comm_ref[k].astype(jnp.float32) - m)
            comm_ref[k] = ek.astype(jnp.bfloat16)
            return s + jnp.sum(ek, axis=1, keepdims=True)

        s = lax.fori_loop(0, N_DEV, exp_body,
                          jnp.zeros((t, 1), dtype=jnp.float32))
        inv = 1.0 / s

        def store_body(k, carry):
            slot = lax.rem(k, 2)
            origin = lax.rem(my - k + N_DEV, N_DEV)
            stage_ref[slot] = comm_ref[k].astype(jnp.float32) * inv
            copy = pltpu.make_async_copy(
                stage_ref.at[slot],
                out_ref.at[:, pl.ds(origin * v_loc, v_loc)],
                out_sems.at[slot],
            )
            copy.start()
            copy.wait()
            return carry

        lax.fori_loop(0, N_DEV, store_body, 0)

    return pl.pallas_call(
        body,
        out_shape=jax.ShapeDtypeStruct((t, v_tot), jnp.float32),
        in_specs=[
            pl.BlockSpec(memory_space=pltpu.VMEM),
            pl.BlockSpec(memory_space=pltpu.VMEM),
        ],
        out_specs=pl.BlockSpec(memory_space=pl.ANY),
        scratch_shapes=[
            pltpu.VMEM((N_DEV, t, v_loc), jnp.bfloat16),
            pltpu.VMEM((2, t, v_loc), jnp.float32),
            pltpu.SemaphoreType.DMA((N_DEV,)),
            pltpu.SemaphoreType.DMA((N_DEV,)),
            pltpu.SemaphoreType.DMA((2,)),
        ],
        compiler_params=pltpu.CompilerParams(
            collective_id=0,
            vmem_limit_bytes=63 * 1024 * 1024,
        ),
    )(x, W)


# device time: 151852 ns/iter; 1.5128x vs baseline; 1.5128x over previous
import jax
import jax.numpy as jnp
from jax import lax
from jax.experimental import pallas as pl
from jax.experimental.pallas import tpu as pltpu

N_DEV = 8
CW_HOPS = 4
CCW_HOPS = 3


def kernel(x, W):
    t, d = x.shape
    _, v_loc = W.shape
    v_tot = N_DEV * v_loc

    def body(x_ref, w_ref, out_ref, comm_ref, stage_ref,
             send_sems, recv_sems, out_sems):
        my = lax.axis_index("i")
        left = lax.rem(my + (N_DEV - 1), N_DEV)
        right = lax.rem(my + 1, N_DEV)

        barrier_sem = pltpu.get_barrier_semaphore()
        for nbr in (left, right):
            pl.semaphore_signal(
                barrier_sem, inc=1,
                device_id=(nbr,), device_id_type=pl.DeviceIdType.MESH,
            )
        pl.semaphore_wait(barrier_sem, 2)

        xb = x_ref[...].astype(jnp.bfloat16)
        wb = w_ref[...].astype(jnp.bfloat16)
        logits = jnp.dot(xb, wb, preferred_element_type=jnp.float32)
        comm_ref[0] = logits.astype(jnp.bfloat16)

        for s in range(1, CW_HOPS + 1):
            cw = pltpu.make_async_remote_copy(
                src_ref=comm_ref.at[s - 1],
                dst_ref=comm_ref.at[s],
                send_sem=send_sems.at[s - 1],
                recv_sem=recv_sems.at[s],
                device_id=(right,),
                device_id_type=pl.DeviceIdType.MESH,
            )
            cw.start()
            ccw = None
            if s <= CCW_HOPS:
                src_slot = 0 if s == 1 else 4 + s - 1
                ccw = pltpu.make_async_remote_copy(
                    src_ref=comm_ref.at[src_slot],
                    dst_ref=comm_ref.at[4 + s],
                    send_sem=send_sems.at[4 + s],
                    recv_sem=recv_sems.at[4 + s],
                    device_id=(left,),
                    device_id_type=pl.DeviceIdType.MESH,
                )
                ccw.start()
            cw.wait()
            if ccw is not None:
                ccw.wait()

        def max_body(k, mb):
            return jnp.maximum(mb, comm_ref[k])

        mb = lax.fori_loop(1, N_DEV, max_body, comm_ref[0])
        m = jnp.max(mb.astype(jnp.float32), axis=1, keepdims=True)

        def exp_body(k, acc):
            ek = jnp.exp(comm_ref[k].astype(jnp.float32) - m)
            comm_ref[k] = ek.astype(jnp.bfloat16)
            return acc + jnp.sum(ek, axis=1, keepdims=True)

        ssum = lax.fori_loop(0, N_DEV, exp_body,
                             jnp.zeros((t, 1), dtype=jnp.float32))
        inv = 1.0 / ssum

        def out_copy(slot, origin):
            return pltpu.make_async_copy(
                stage_ref.at[slot],
                out_ref.at[:, pl.ds(origin * v_loc, v_loc)],
                out_sems.at[slot],
            )

        def origin_of(k):
            o = jnp.where(k <= CW_HOPS, my - k, my + k - CW_HOPS)
            return lax.rem(o + N_DEV, N_DEV)

        def store_body(k, carry):
            slot = lax.rem(k, 2)

            @pl.when(k >= 2)
            def _():
                out_copy(slot, origin_of(k - 2)).wait()

            stage_ref[slot] = comm_ref[k].astype(jnp.float32) * inv
            out_copy(slot, origin_of(k)).start()
            return carry

        lax.fori_loop(0, N_DEV, store_body, 0)
        out_copy(0, origin_of(N_DEV - 2)).wait()
        out_copy(1, origin_of(N_DEV - 1)).wait()

    return pl.pallas_call(
        body,
        out_shape=jax.ShapeDtypeStruct((t, v_tot), jnp.float32),
        in_specs=[
            pl.BlockSpec(memory_space=pltpu.VMEM),
            pl.BlockSpec(memory_space=pltpu.VMEM),
        ],
        out_specs=pl.BlockSpec(memory_space=pl.ANY),
        scratch_shapes=[
            pltpu.VMEM((N_DEV, t, v_loc), jnp.bfloat16),
            pltpu.VMEM((2, t, v_loc), jnp.float32),
            pltpu.SemaphoreType.DMA((N_DEV,)),
            pltpu.SemaphoreType.DMA((N_DEV,)),
            pltpu.SemaphoreType.DMA((2,)),
        ],
        compiler_params=pltpu.CompilerParams(
            collective_id=0,
            vmem_limit_bytes=63 * 1024 * 1024,
        ),
    )(x, W)


# device time: 132300 ns/iter; 1.7363x vs baseline; 1.1478x over previous
import jax
import jax.numpy as jnp
from jax import lax
from jax.experimental import pallas as pl
from jax.experimental.pallas import tpu as pltpu

N_DEV = 8


def kernel(x, W):
    t, d = x.shape
    _, v_loc = W.shape
    v_tot = N_DEV * v_loc
    v_half = v_loc // 2

    def body(x_ref, w_ref, out_ref, comm_ref, stage_ref,
             send_sems, recv_sems, out_sems):
        my = lax.axis_index("i")
        left = lax.rem(my + (N_DEV - 1), N_DEV)
        right = lax.rem(my + 1, N_DEV)

        barrier_sem = pltpu.get_barrier_semaphore()
        for nbr in (left, right):
            pl.semaphore_signal(
                barrier_sem, inc=1,
                device_id=(nbr,), device_id_type=pl.DeviceIdType.MESH,
            )
        pl.semaphore_wait(barrier_sem, 2)

        xb = x_ref[...].astype(jnp.bfloat16)
        wb = w_ref[...].astype(jnp.bfloat16)
        logits = jnp.dot(xb, wb, preferred_element_type=jnp.float32)
        comm_ref[0] = logits.astype(jnp.bfloat16)

        def cw_rdma(s):
            if s < 4:
                src, dst = comm_ref.at[s - 1], comm_ref.at[s]
            else:
                src = comm_ref.at[3, :, 0:v_half]
                dst = comm_ref.at[4, :, 0:v_half]
            return pltpu.make_async_remote_copy(
                src_ref=src, dst_ref=dst,
                send_sem=send_sems.at[s - 1], recv_sem=recv_sems.at[s],
                device_id=(right,), device_id_type=pl.DeviceIdType.MESH,
            )

        def ccw_rdma(s):
            if s == 1:
                src, dst = comm_ref.at[0], comm_ref.at[5]
            elif s < 4:
                src, dst = comm_ref.at[4 + s - 1], comm_ref.at[4 + s]
            else:
                src = comm_ref.at[7, :, v_half:v_loc]
                dst = comm_ref.at[4, :, v_half:v_loc]
            recv = recv_sems.at[4 + s] if s < 4 else recv_sems.at[8]
            return pltpu.make_async_remote_copy(
                src_ref=src, dst_ref=dst,
                send_sem=send_sems.at[3 + s], recv_sem=recv,
                device_id=(left,), device_id_type=pl.DeviceIdType.MESH,
            )

        def online(state, k):
            m, ssum = state
            c = comm_ref[k].astype(jnp.float32)
            m_new = jnp.maximum(m, jnp.max(c, axis=1, keepdims=True))
            ssum = ssum * jnp.exp(m - m_new) + jnp.sum(
                jnp.exp(c - m_new), axis=1, keepdims=True
            )
            return m_new, ssum

        c0 = comm_ref[0].astype(jnp.float32)
        state = (
            jnp.max(c0, axis=1, keepdims=True),
            jnp.zeros((t, 1), dtype=jnp.float32),
        )
        state = (state[0], jnp.sum(jnp.exp(c0 - state[0]), axis=1,
                                   keepdims=True))
        for s in range(1, 5):
            cw = cw_rdma(s)
            ccw = ccw_rdma(s)
            cw.start()
            ccw.start()
            if s >= 2:
                state = online(state, s - 1)
                state = online(state, 4 + s - 1)
            cw.wait()
            ccw.wait()
        state = online(state, 4)
        m, ssum = state
        shift = m + jnp.log(ssum)

        def out_copy(slot, origin):
            return pltpu.make_async_copy(
                stage_ref.at[slot],
                out_ref.at[:, pl.ds(origin * v_loc, v_loc)],
                out_sems.at[slot],
            )

        def origin_of(k):
            o = jnp.where(k <= 4, my - k, my + k - 4)
            return lax.rem(o + N_DEV, N_DEV)

        def store_body(k, carry):
            slot = lax.rem(k, 2)

            @pl.when(k >= 2)
            def _():
                out_copy(slot, origin_of(k - 2)).wait()

            stage_ref[slot] = jnp.exp(
                comm_ref[k].astype(jnp.float32) - shift
            )
            out_copy(slot, origin_of(k)).start()
            return carry

        lax.fori_loop(0, N_DEV, store_body, 0)
        out_copy(0, origin_of(N_DEV - 2)).wait()
        out_copy(1, origin_of(N_DEV - 1)).wait()

    return pl.pallas_call(
        body,
        out_shape=jax.ShapeDtypeStruct((t, v_tot), jnp.float32),
        in_specs=[
            pl.BlockSpec(memory_space=pltpu.VMEM),
            pl.BlockSpec(memory_space=pltpu.VMEM),
        ],
        out_specs=pl.BlockSpec(memory_space=pl.ANY),
        scratch_shapes=[
            pltpu.VMEM((N_DEV, t, v_loc), jnp.bfloat16),
            pltpu.VMEM((2, t, v_loc), jnp.float32),
            pltpu.SemaphoreType.DMA((8,)),
            pltpu.SemaphoreType.DMA((9,)),
            pltpu.SemaphoreType.DMA((2,)),
        ],
        compiler_params=pltpu.CompilerParams(
            collective_id=0,
            vmem_limit_bytes=63 * 1024 * 1024,
        ),
    )(x, W)


# device time: 94149 ns/iter; 2.4399x vs baseline; 1.4052x over previous
import jax
import jax.numpy as jnp
from jax import lax
from jax.experimental import pallas as pl
from jax.experimental.pallas import tpu as pltpu

N_DEV = 8
NC = 4


def kernel(x, W):
    t, d = x.shape
    _, v_loc = W.shape
    v_tot = N_DEV * v_loc
    v_c = v_loc // NC

    def body(x_ref, w_ref, out_ref, comm_ref, stage_ref,
             send_sems, recv_sems, out_sems):
        my = lax.axis_index("i")

        def P(i):
            return jnp.where(i <= 3, i, 11 - i)

        def modN(i):
            return lax.rem(i + 2 * N_DEV, N_DEV)

        r = P(my)
        right = P(modN(r + 1))
        left = P(modN(r - 1))
        is_even = lax.rem(r, 2) == 0
        chord = P(modN(jnp.where(is_even, r + 3, r - 3)))

        barrier_sem = pltpu.get_barrier_semaphore()
        for nbr in (left, right, chord):
            pl.semaphore_signal(
                barrier_sem, inc=1,
                device_id=(nbr,), device_id_type=pl.DeviceIdType.MESH,
            )
        pl.semaphore_wait(barrier_sem, 3)

        xb = x_ref[...].astype(jnp.bfloat16)

        def gemm_cell(c):
            cols = pl.ds(c * v_c, v_c)
            wb = w_ref[:, cols].astype(jnp.bfloat16)
            lg = jnp.dot(xb, wb, preferred_element_type=jnp.float32)
            comm_ref[0, :, cols] = lg.astype(jnp.bfloat16)

        def rdma(src_slot, dst_slot, send_idx, cell, target):
            cols = pl.ds(cell * v_c, v_c)
            return pltpu.make_async_remote_copy(
                src_ref=comm_ref.at[src_slot, :, cols],
                dst_ref=comm_ref.at[dst_slot, :, cols],
                send_sem=send_sems.at[send_idx, cell],
                recv_sem=recv_sems.at[dst_slot, cell],
                device_id=(target,),
                device_id_type=pl.DeviceIdType.MESH,
            )

        def online(state, k):
            m, ssum = state
            c = comm_ref[k].astype(jnp.float32)
            m_new = jnp.maximum(m, jnp.max(c, axis=1, keepdims=True))
            ssum = ssum * jnp.exp(m - m_new) + jnp.sum(
                jnp.exp(c - m_new), axis=1, keepdims=True
            )
            return m_new, ssum

        ch1_dst = jnp.where(is_even, 3, 7)
        ch1_arr = jnp.where(is_even, 7, 3)
        ch2_src = jnp.where(is_even, 1, 5)
        ch3_src = jnp.where(is_even, 2, 6)

        cw1 = [rdma(0, 1, 0, c, right) for c in range(NC)]
        ccw1 = [rdma(0, 5, 3, c, left) for c in range(NC)]
        ch1 = [rdma(0, ch1_dst, 6, c, chord) for c in range(NC)]
        for c in range(NC):
            gemm_cell(c)
            cw1[c].start()
            ccw1[c].start()
            ch1[c].start()
        state = online(
            (jnp.full((t, 1), -jnp.inf, jnp.float32),
             jnp.zeros((t, 1), jnp.float32)),
            0,
        )

        cw2 = [rdma(1, 2, 1, c, right) for c in range(NC)]
        ccw2 = [rdma(5, 6, 4, c, left) for c in range(NC)]
        ch2 = [rdma(ch2_src, 4, 7, c, chord) for c in range(NC)]
        for c in range(NC):
            cw1[c].wait_recv()
            cw2[c].start()
            ccw1[c].wait_recv()
            ccw2[c].start()
            ch2[c].start()
        state = online(state, 1)
        state = online(state, 5)

        cw3 = [rdma(2, 3, 2, c, right) for c in (0, 1)]
        ccw3 = [rdma(6, 7, 5, c, left) for c in (0, 1)]
        ch3 = {c: rdma(ch3_src, ch1_arr, 2, c, chord) for c in (2, 3)}
        for c in (0, 1):
            cw2[c].wait_recv()

            @pl.when(jnp.logical_not(is_even))
            def _():
                cw3[c].start()

            ccw2[c].wait_recv()

            @pl.when(is_even)
            def _():
                ccw3[c].start()

        for c in (2, 3):
            cw2[c].wait_recv()
            ccw2[c].wait_recv()
            ch3[c].start()

        state = online(state, 2)
        state = online(state, 6)

        for c in range(NC):
            rdma(0, ch1_arr, 6, c, chord).wait_recv()
        state = online(state, ch1_arr)
        for c in range(NC):
            ch2[c].wait_recv()
        state = online(state, 4)
        for c in range(NC):
            rdma(0, ch1_dst, 2, c, chord).wait_recv()
        state = online(state, ch1_dst)

        for rr in cw1 + ccw1 + ch1 + cw2 + ccw2 + ch2 + list(ch3.values()):
            rr.wait_send()
        for c in (0, 1):
            @pl.when(jnp.logical_not(is_even))
            def _():
                cw3[c].wait_send()

            @pl.when(is_even)
            def _():
                ccw3[c].wait_send()

        m, ssum = state
        shift = m + jnp.log(ssum)

        def out_copy(slot, origin):
            return pltpu.make_async_copy(
                stage_ref.at[slot],
                out_ref.at[:, pl.ds(origin * v_loc, v_loc)],
                out_sems.at[slot],
            )

        def origin_of(k):
            off = jnp.where(
                k == 0, 0,
                jnp.where(k <= 3, -k, jnp.where(k == 4, 4, k - 4)),
            )
            return P(modN(r + off))

        def store_body(k, carry):
            slot = lax.rem(k, 2)

            @pl.when(k >= 2)
            def _():
                out_copy(slot, origin_of(k - 2)).wait()

            stage_ref[slot] = jnp.exp(
                comm_ref[k].astype(jnp.float32) - shift
            ).astype(jnp.bfloat16)
            out_copy(slot, origin_of(k)).start()
            return carry

        lax.fori_loop(0, N_DEV, store_body, 0)
        out_copy(0, origin_of(N_DEV - 2)).wait()
        out_copy(1, origin_of(N_DEV - 1)).wait()

    return pl.pallas_call(
        body,
        out_shape=jax.ShapeDtypeStruct((t, v_tot), jnp.bfloat16),
        in_specs=[
            pl.BlockSpec(memory_space=pltpu.VMEM),
            pl.BlockSpec(memory_space=pltpu.VMEM),
        ],
        out_specs=pl.BlockSpec(memory_space=pl.ANY),
        scratch_shapes=[
            pltpu.VMEM((N_DEV, t, v_loc), jnp.bfloat16),
            pltpu.VMEM((2, t, v_loc), jnp.bfloat16),
            pltpu.SemaphoreType.DMA((8, NC)),
            pltpu.SemaphoreType.DMA((8, NC)),
            pltpu.SemaphoreType.DMA((2,)),
        ],
        compiler_params=pltpu.CompilerParams(
            collective_id=0,
            vmem_limit_bytes=63 * 1024 * 1024,
        ),
    )(x, W).astype(jnp.float32)
